# Initial kernel scaffold; baseline (speedup 1.0000x reference)
import jax
import jax.numpy as jnp
from jax.experimental import pallas as pl


def kernel(x, emb_table, emb_small_table, Wc, bc, gc, betac, W1, b1, g1, be1, W2, b2, g2, be2, W3, b3, g3, be3, Wo, bo):
    raise NotImplementedError("write your pallas kernel here")


# speedup vs baseline: 2.8619x; 2.8619x over previous
import functools

import jax
import jax.numpy as jnp
from jax import lax
from jax.experimental import pallas as pl
from jax.experimental.pallas import tpu as pltpu
from jax.experimental.pallas import tpu_sc as plsc

B = 4096
F = 26
PER = 4000
D = 64
DS = 16
K = 13
NROWS = B * F
H1, H2, H3 = 1024, 512, 256
EPS = 1e-5

NC, NS = 2, 16
NW = NC * NS
PW = NROWS // NW
SUB = 128
NSUB = PW // SUB

TB = 512
NB = B // TB


def _sc_gather(xi, small_tbl, main_tbl):
  mesh = plsc.VectorSubcoreMesh(core_axis_name="c", subcore_axis_name="s")

  @functools.partial(
      pl.kernel,
      out_type=(
          jax.ShapeDtypeStruct((NROWS, DS), jnp.float32),
          jax.ShapeDtypeStruct((NROWS, D), jnp.float32),
      ),
      mesh=mesh,
      compiler_params=pltpu.CompilerParams(use_tc_tiling_on_sc=False),
      scratch_types=[
          pltpu.VMEM((SUB,), jnp.int32),
          pltpu.VMEM((SUB, DS), jnp.float32),
          pltpu.VMEM((SUB, D), jnp.float32),
          pltpu.SemaphoreType.DMA,
          pltpu.SemaphoreType.DMA,
      ],
  )
  def k(xi_hbm, sm_hbm, mn_hbm, es_out, em_out, idx_v, sm_v, mn_v, sem1, sem2):
    wid = lax.axis_index("s") * NC + lax.axis_index("c")
    base0 = wid * PW

    def body(j, carry):
      base = base0 + j * SUB
      pltpu.sync_copy(xi_hbm.at[pl.ds(base, SUB)], idx_v)
      c1 = pltpu.async_copy(sm_hbm.at[idx_v], sm_v, sem1)
      c2 = pltpu.async_copy(mn_hbm.at[idx_v], mn_v, sem2)
      c1.wait()
      c2.wait()
      pltpu.sync_copy(sm_v, es_out.at[pl.ds(base, SUB)])
      pltpu.sync_copy(mn_v, em_out.at[pl.ds(base, SUB)])
      return carry

    lax.fori_loop(0, NSUB, body, 0)

  return k(xi, small_tbl, main_tbl)


def _controller(es2, wcp, bc, gc, betac):
  def body(es_ref, w_ref, bc_ref, gc_ref, be_ref, ms_ref):
    y = jnp.dot(es_ref[...], w_ref[...],
                preferred_element_type=jnp.float32) + bc_ref[...]
    mean = jnp.mean(y, axis=0, keepdims=True)
    var = jnp.mean((y - mean) ** 2, axis=0, keepdims=True)
    h = jnp.maximum(
        gc_ref[...] * (y - mean) / jnp.sqrt(var + EPS) + be_ref[...], 0.0)
    m = jnp.max(h, axis=1, keepdims=True)
    e = jnp.exp(h - m)
    s = e / jnp.sum(e, axis=1, keepdims=True)
    iota_f = lax.broadcasted_iota(jnp.int32, (B, F), 1)
    cnt = jnp.zeros((B, F), jnp.float32)
    for g in range(F):
      sg = s[:, g:g + 1]
      beats = (sg > s) | ((sg == s) & (g < iota_f))
      cnt = cnt + jnp.where(beats, 1.0, 0.0)
    ms_ref[...] = jnp.where(cnt < K, s, 0.0)

  return pl.pallas_call(
      body,
      out_shape=jax.ShapeDtypeStruct((B, F), jnp.float32),
  )(es2, wcp, bc, gc, betac)


def _layer1(em2, ms, expand, w1p, b1):
  def body(em_ref, ms_ref, e_ref, w_ref, b_ref, y_ref, s_ref, q_ref):
    i = pl.program_id(0)
    msx = jnp.dot(ms_ref[...], e_ref[...], preferred_element_type=jnp.float32)
    z = em_ref[...] * msx
    y = jnp.dot(z, w_ref[...], preferred_element_type=jnp.float32) + b_ref[...]
    y_ref[...] = y

    @pl.when(i == 0)
    def _():
      s_ref[...] = jnp.zeros_like(s_ref)
      q_ref[...] = jnp.zeros_like(q_ref)

    s_ref[...] += jnp.sum(y, axis=0, keepdims=True)
    q_ref[...] += jnp.sum(y * y, axis=0, keepdims=True)

  return pl.pallas_call(
      body,
      grid=(NB,),
      in_specs=[
          pl.BlockSpec((TB, F * D), lambda i: (i, 0)),
          pl.BlockSpec((TB, F), lambda i: (i, 0)),
          pl.BlockSpec((F, F * D), lambda i: (0, 0)),
          pl.BlockSpec((F * D, H1), lambda i: (0, 0)),
          pl.BlockSpec((1, H1), lambda i: (0, 0)),
      ],
      out_specs=[
          pl.BlockSpec((TB, H1), lambda i: (i, 0)),
          pl.BlockSpec((1, H1), lambda i: (0, 0)),
          pl.BlockSpec((1, H1), lambda i: (0, 0)),
      ],
      out_shape=[
          jax.ShapeDtypeStruct((B, H1), jnp.float32),
          jax.ShapeDtypeStruct((1, H1), jnp.float32),
          jax.ShapeDtypeStruct((1, H1), jnp.float32),
      ],
  )(em2, ms, expand, w1p, b1)


def _mid_layer(y_in, s_in, q_in, g, be, w, b, h_in, h_out):
  def body(y_ref, s_ref, q_ref, g_ref, be_ref, w_ref, b_ref,
           yo_ref, so_ref, qo_ref):
    i = pl.program_id(0)
    mean = s_ref[...] * (1.0 / B)
    var = q_ref[...] * (1.0 / B) - mean * mean
    h = jnp.maximum(
        g_ref[...] * (y_ref[...] - mean) / jnp.sqrt(var + EPS) + be_ref[...],
        0.0)
    y = jnp.dot(h, w_ref[...], preferred_element_type=jnp.float32) + b_ref[...]
    yo_ref[...] = y

    @pl.when(i == 0)
    def _():
      so_ref[...] = jnp.zeros_like(so_ref)
      qo_ref[...] = jnp.zeros_like(qo_ref)

    so_ref[...] += jnp.sum(y, axis=0, keepdims=True)
    qo_ref[...] += jnp.sum(y * y, axis=0, keepdims=True)

  return pl.pallas_call(
      body,
      grid=(NB,),
      in_specs=[
          pl.BlockSpec((TB, h_in), lambda i: (i, 0)),
          pl.BlockSpec((1, h_in), lambda i: (0, 0)),
          pl.BlockSpec((1, h_in), lambda i: (0, 0)),
          pl.BlockSpec((1, h_in), lambda i: (0, 0)),
          pl.BlockSpec((1, h_in), lambda i: (0, 0)),
          pl.BlockSpec((h_in, h_out), lambda i: (0, 0)),
          pl.BlockSpec((1, h_out), lambda i: (0, 0)),
      ],
      out_specs=[
          pl.BlockSpec((TB, h_out), lambda i: (i, 0)),
          pl.BlockSpec((1, h_out), lambda i: (0, 0)),
          pl.BlockSpec((1, h_out), lambda i: (0, 0)),
      ],
      out_shape=[
          jax.ShapeDtypeStruct((B, h_out), jnp.float32),
          jax.ShapeDtypeStruct((1, h_out), jnp.float32),
          jax.ShapeDtypeStruct((1, h_out), jnp.float32),
      ],
  )(y_in, s_in, q_in, g, be, w, b)


def _final_layer(y_in, s_in, q_in, g, be, wo, bo):
  def body(y_ref, s_ref, q_ref, g_ref, be_ref, w_ref, b_ref, o_ref):
    mean = s_ref[...] * (1.0 / B)
    var = q_ref[...] * (1.0 / B) - mean * mean
    h = jnp.maximum(
        g_ref[...] * (y_ref[...] - mean) / jnp.sqrt(var + EPS) + be_ref[...],
        0.0)
    t = jnp.dot(h, w_ref[...], preferred_element_type=jnp.float32) + b_ref[...]
    o_ref[...] = jax.nn.sigmoid(t)

  return pl.pallas_call(
      body,
      grid=(NB,),
      in_specs=[
          pl.BlockSpec((TB, H3), lambda i: (i, 0)),
          pl.BlockSpec((1, H3), lambda i: (0, 0)),
          pl.BlockSpec((1, H3), lambda i: (0, 0)),
          pl.BlockSpec((1, H3), lambda i: (0, 0)),
          pl.BlockSpec((1, H3), lambda i: (0, 0)),
          pl.BlockSpec((H3, 1), lambda i: (0, 0)),
          pl.BlockSpec((1, 1), lambda i: (0, 0)),
      ],
      out_specs=pl.BlockSpec((TB, 1), lambda i: (i, 0)),
      out_shape=jax.ShapeDtypeStruct((B, 1), jnp.float32),
  )(y_in, s_in, q_in, g, be, wo, bo)


def kernel(x, emb_table, emb_small_table, Wc, bc, gc, betac,
           W1, b1, g1, be1, W2, b2, g2, be2, W3, b3, g3, be3, Wo, bo):
  offs = (jnp.arange(F, dtype=jnp.int32) * PER).astype(x.dtype)
  xi = (x + offs[None, :]).reshape(-1).astype(jnp.int32)

  es, em = _sc_gather(xi, emb_small_table, emb_table)

  es2 = es.reshape(B, F * DS)
  wcp = Wc.reshape(DS, F, F).transpose(1, 0, 2).reshape(F * DS, F)
  ms = _controller(es2, wcp, bc.reshape(1, F), gc.reshape(1, F),
                   betac.reshape(1, F))

  em2 = em.reshape(B, F * D)
  w1p = W1.reshape(D, F, H1).transpose(1, 0, 2).reshape(F * D, H1)
  expand = (jnp.arange(F)[:, None] ==
            (jnp.arange(F * D)[None, :] // D)).astype(jnp.float32)

  y1, s1, q1 = _layer1(em2, ms, expand, w1p, b1.reshape(1, H1))
  y2, s2, q2 = _mid_layer(y1, s1, q1, g1.reshape(1, H1), be1.reshape(1, H1),
                          W2, b2.reshape(1, H2), H1, H2)
  y3, s3, q3 = _mid_layer(y2, s2, q2, g2.reshape(1, H2), be2.reshape(1, H2),
                          W3, b3.reshape(1, H3), H2, H3)
  out = _final_layer(y3, s3, q3, g3.reshape(1, H3), be3.reshape(1, H3),
                     Wo, bo.reshape(1, 1))
  return out


# speedup vs baseline: 3.0771x; 1.0752x over previous
import functools

import jax
import jax.numpy as jnp
from jax import lax
from jax.experimental import pallas as pl
from jax.experimental.pallas import tpu as pltpu
from jax.experimental.pallas import tpu_sc as plsc

B = 4096
F = 26
PER = 4000
D = 64
DS = 16
K = 13
NROWS = B * F
H1, H2, H3 = 1024, 512, 256
EPS = 1e-5

NC, NS = 2, 16
NW = NC * NS
PW = NROWS // NW
SUB = 128
NSUB = PW // SUB

TB = 512
NB = B // TB


def _sc_gather(xi, small_tbl, main_tbl):
  mesh = plsc.VectorSubcoreMesh(core_axis_name="c", subcore_axis_name="s")

  @functools.partial(
      pl.kernel,
      out_type=(
          jax.ShapeDtypeStruct((NROWS, DS), jnp.float32),
          jax.ShapeDtypeStruct((NROWS, D), jnp.float32),
      ),
      mesh=mesh,
      compiler_params=pltpu.CompilerParams(use_tc_tiling_on_sc=False),
      scratch_types=[
          pltpu.VMEM((SUB,), jnp.int32),
          pltpu.VMEM((SUB, DS), jnp.float32),
          pltpu.VMEM((SUB, D), jnp.float32),
          pltpu.SemaphoreType.DMA,
          pltpu.SemaphoreType.DMA,
      ],
  )
  def k(xi_hbm, sm_hbm, mn_hbm, es_out, em_out, idx_v, sm_v, mn_v, sem1, sem2):
    wid = lax.axis_index("s") * NC + lax.axis_index("c")
    base0 = wid * PW

    def body(j, carry):
      base = base0 + j * SUB
      pltpu.sync_copy(xi_hbm.at[pl.ds(base, SUB)], idx_v)
      c1 = pltpu.async_copy(sm_hbm.at[idx_v], sm_v, sem1)
      c2 = pltpu.async_copy(mn_hbm.at[idx_v], mn_v, sem2)
      c1.wait()
      c2.wait()
      pltpu.sync_copy(sm_v, es_out.at[pl.ds(base, SUB)])
      pltpu.sync_copy(mn_v, em_out.at[pl.ds(base, SUB)])
      return carry

    lax.fori_loop(0, NSUB, body, 0)

  return k(xi, small_tbl, main_tbl)


def _controller(es2, wcp, bc, gc, betac):
  def body(es_ref, w_ref, bc_ref, gc_ref, be_ref, ms_ref):
    y = jnp.dot(es_ref[...], w_ref[...],
                preferred_element_type=jnp.float32) + bc_ref[...]
    mean = jnp.mean(y, axis=0, keepdims=True)
    var = jnp.mean((y - mean) ** 2, axis=0, keepdims=True)
    h = jnp.maximum(
        gc_ref[...] * (y - mean) / jnp.sqrt(var + EPS) + be_ref[...], 0.0)
    m = jnp.max(h, axis=1, keepdims=True)
    e = jnp.exp(h - m)
    s = e / jnp.sum(e, axis=1, keepdims=True)
    sT = s.T
    iota_f = lax.broadcasted_iota(jnp.int32, (F, B), 0)
    cntT = jnp.zeros((F, B), jnp.float32)
    for g in range(F):
      sg = sT[g:g + 1, :]
      beats = (sg > sT) | ((sg == sT) & (iota_f > g))
      cntT = cntT + jnp.where(beats, 1.0, 0.0)
    msT = jnp.where(cntT < K, sT, 0.0)
    ms_ref[...] = msT.T

  return pl.pallas_call(
      body,
      out_shape=jax.ShapeDtypeStruct((B, F), jnp.float32),
  )(es2, wcp, bc, gc, betac)


def _layer1(em2, ms, expand, w1p, b1):
  def body(em_ref, ms_ref, e_ref, w_ref, b_ref, y_ref, s_ref, q_ref):
    i = pl.program_id(0)
    msx = jnp.dot(ms_ref[...], e_ref[...], preferred_element_type=jnp.float32)
    z = em_ref[...] * msx
    y = jnp.dot(z, w_ref[...], preferred_element_type=jnp.float32) + b_ref[...]
    y_ref[...] = y

    @pl.when(i == 0)
    def _():
      s_ref[...] = jnp.zeros_like(s_ref)
      q_ref[...] = jnp.zeros_like(q_ref)

    s_ref[...] += jnp.sum(y, axis=0, keepdims=True)
    q_ref[...] += jnp.sum(y * y, axis=0, keepdims=True)

  return pl.pallas_call(
      body,
      grid=(NB,),
      in_specs=[
          pl.BlockSpec((TB, F * D), lambda i: (i, 0)),
          pl.BlockSpec((TB, F), lambda i: (i, 0)),
          pl.BlockSpec((F, F * D), lambda i: (0, 0)),
          pl.BlockSpec((F * D, H1), lambda i: (0, 0)),
          pl.BlockSpec((1, H1), lambda i: (0, 0)),
      ],
      out_specs=[
          pl.BlockSpec((TB, H1), lambda i: (i, 0)),
          pl.BlockSpec((1, H1), lambda i: (0, 0)),
          pl.BlockSpec((1, H1), lambda i: (0, 0)),
      ],
      out_shape=[
          jax.ShapeDtypeStruct((B, H1), jnp.float32),
          jax.ShapeDtypeStruct((1, H1), jnp.float32),
          jax.ShapeDtypeStruct((1, H1), jnp.float32),
      ],
  )(em2, ms, expand, w1p, b1)


def _mid_layer(y_in, s_in, q_in, g, be, w, b, h_in, h_out):
  def body(y_ref, s_ref, q_ref, g_ref, be_ref, w_ref, b_ref,
           yo_ref, so_ref, qo_ref):
    i = pl.program_id(0)
    mean = s_ref[...] * (1.0 / B)
    var = q_ref[...] * (1.0 / B) - mean * mean
    h = jnp.maximum(
        g_ref[...] * (y_ref[...] - mean) / jnp.sqrt(var + EPS) + be_ref[...],
        0.0)
    y = jnp.dot(h, w_ref[...], preferred_element_type=jnp.float32) + b_ref[...]
    yo_ref[...] = y

    @pl.when(i == 0)
    def _():
      so_ref[...] = jnp.zeros_like(so_ref)
      qo_ref[...] = jnp.zeros_like(qo_ref)

    so_ref[...] += jnp.sum(y, axis=0, keepdims=True)
    qo_ref[...] += jnp.sum(y * y, axis=0, keepdims=True)

  return pl.pallas_call(
      body,
      grid=(NB,),
      in_specs=[
          pl.BlockSpec((TB, h_in), lambda i: (i, 0)),
          pl.BlockSpec((1, h_in), lambda i: (0, 0)),
          pl.BlockSpec((1, h_in), lambda i: (0, 0)),
          pl.BlockSpec((1, h_in), lambda i: (0, 0)),
          pl.BlockSpec((1, h_in), lambda i: (0, 0)),
          pl.BlockSpec((h_in, h_out), lambda i: (0, 0)),
          pl.BlockSpec((1, h_out), lambda i: (0, 0)),
      ],
      out_specs=[
          pl.BlockSpec((TB, h_out), lambda i: (i, 0)),
          pl.BlockSpec((1, h_out), lambda i: (0, 0)),
          pl.BlockSpec((1, h_out), lambda i: (0, 0)),
      ],
      out_shape=[
          jax.ShapeDtypeStruct((B, h_out), jnp.float32),
          jax.ShapeDtypeStruct((1, h_out), jnp.float32),
          jax.ShapeDtypeStruct((1, h_out), jnp.float32),
      ],
  )(y_in, s_in, q_in, g, be, w, b)


def _final_layer(y_in, s_in, q_in, g, be, wo, bo):
  def body(y_ref, s_ref, q_ref, g_ref, be_ref, w_ref, b_ref, o_ref):
    mean = s_ref[...] * (1.0 / B)
    var = q_ref[...] * (1.0 / B) - mean * mean
    h = jnp.maximum(
        g_ref[...] * (y_ref[...] - mean) / jnp.sqrt(var + EPS) + be_ref[...],
        0.0)
    t = jnp.dot(h, w_ref[...], preferred_element_type=jnp.float32) + b_ref[...]
    o_ref[...] = jax.nn.sigmoid(t)

  return pl.pallas_call(
      body,
      grid=(NB,),
      in_specs=[
          pl.BlockSpec((TB, H3), lambda i: (i, 0)),
          pl.BlockSpec((1, H3), lambda i: (0, 0)),
          pl.BlockSpec((1, H3), lambda i: (0, 0)),
          pl.BlockSpec((1, H3), lambda i: (0, 0)),
          pl.BlockSpec((1, H3), lambda i: (0, 0)),
          pl.BlockSpec((H3, 1), lambda i: (0, 0)),
          pl.BlockSpec((1, 1), lambda i: (0, 0)),
      ],
      out_specs=pl.BlockSpec((TB, 1), lambda i: (i, 0)),
      out_shape=jax.ShapeDtypeStruct((B, 1), jnp.float32),
  )(y_in, s_in, q_in, g, be, wo, bo)


def kernel(x, emb_table, emb_small_table, Wc, bc, gc, betac,
           W1, b1, g1, be1, W2, b2, g2, be2, W3, b3, g3, be3, Wo, bo):
  offs = (jnp.arange(F, dtype=jnp.int32) * PER).astype(x.dtype)
  xi = (x + offs[None, :]).reshape(-1).astype(jnp.int32)

  es, em = _sc_gather(xi, emb_small_table, emb_table)

  es2 = es.reshape(B, F * DS)
  wcp = Wc.reshape(DS, F, F).transpose(1, 0, 2).reshape(F * DS, F)
  ms = _controller(es2, wcp, bc.reshape(1, F), gc.reshape(1, F),
                   betac.reshape(1, F))

  em2 = em.reshape(B, F * D)
  w1p = W1.reshape(D, F, H1).transpose(1, 0, 2).reshape(F * D, H1)
  expand = (jnp.arange(F)[:, None] ==
            (jnp.arange(F * D)[None, :] // D)).astype(jnp.float32)

  y1, s1, q1 = _layer1(em2, ms, expand, w1p, b1.reshape(1, H1))
  y2, s2, q2 = _mid_layer(y1, s1, q1, g1.reshape(1, H1), be1.reshape(1, H1),
                          W2, b2.reshape(1, H2), H1, H2)
  y3, s3, q3 = _mid_layer(y2, s2, q2, g2.reshape(1, H2), be2.reshape(1, H2),
                          W3, b3.reshape(1, H3), H2, H3)
  out = _final_layer(y3, s3, q3, g3.reshape(1, H3), be3.reshape(1, H3),
                     Wo, bo.reshape(1, 1))
  return out


# speedup vs baseline: 4.2647x; 1.3859x over previous
import functools

import jax
import jax.numpy as jnp
from jax import lax
from jax.experimental import pallas as pl
from jax.experimental.pallas import tpu as pltpu
from jax.experimental.pallas import tpu_sc as plsc

B = 4096
F = 26
PER = 4000
D = 64
DS = 16
K = 13
H1, H2, H3 = 1024, 512, 256
EPS = 1e-5
NCB = F // 2
NQ = 4

NC, NS = 2, 16
NW = NC * NS
BW = B // NW

TB = 512
NB = B // TB


def _sc_gather(xi_t, small_tbl, main_tbl):
  mesh = plsc.VectorSubcoreMesh(core_axis_name="c", subcore_axis_name="s")
  out_t = tuple(jax.ShapeDtypeStruct((B, 128), jnp.float32)
                for _ in range(NCB + NQ))

  @functools.partial(
      pl.kernel,
      out_type=out_t,
      mesh=mesh,
      compiler_params=pltpu.CompilerParams(use_tc_tiling_on_sc=False),
      scratch_types=[
          pltpu.VMEM((F, BW), jnp.int32),
          pltpu.VMEM((4, BW, D), jnp.float32),
          pltpu.VMEM((4, BW, DS), jnp.float32),
          pltpu.SemaphoreType.DMA((16,)),
      ],
  )
  def k(xi_hbm, sm_hbm, mn_hbm, *rest):
    outs = rest[:NCB + NQ]
    idx_all, mn_v, sm_v, sems = rest[NCB + NQ:]

    wid = lax.axis_index("s") * NC + lax.axis_index("c")
    b0 = wid * BW
    pltpu.sync_copy(xi_hbm.at[:, pl.ds(b0, BW)], idx_all)

    def fire_gathers(f):
      p = f % 4
      g1 = pltpu.async_copy(mn_hbm.at[idx_all.at[f]], mn_v.at[p],
                            sems.at[p])
      g2 = pltpu.async_copy(sm_hbm.at[idx_all.at[f]], sm_v.at[p],
                            sems.at[4 + p])
      return g1, g2

    def fire_scatters(f):
      p = f % 4
      s1 = pltpu.async_copy(
          mn_v.at[p],
          outs[f // 2].at[pl.ds(b0, BW), pl.ds(64 * (f % 2), D)],
          sems.at[8 + p])
      s2 = pltpu.async_copy(
          sm_v.at[p],
          outs[NCB + f // 8].at[pl.ds(b0, BW), pl.ds(16 * (f % 8), DS)],
          sems.at[12 + p])
      return s1, s2

    gat = {0: fire_gathers(0), 1: fire_gathers(1)}
    sca = {}
    for f in range(F):
      g1, g2 = gat.pop(f)
      g1.wait()
      g2.wait()
      sca[f] = fire_scatters(f)
      if f >= 2:
        s1, s2 = sca.pop(f - 2)
        s1.wait()
        s2.wait()
      if f + 2 < F:
        gat[f + 2] = fire_gathers(f + 2)
    for f in (F - 2, F - 1):
      s1, s2 = sca.pop(f)
      s1.wait()
      s2.wait()

  return k(xi_t, small_tbl, main_tbl)


def _controller(es_list, colmask, wcp, bc, gc, betac):
  def body(e0, e1, e2, e3, mk_ref, w_ref, bc_ref, gc_ref, be_ref, ms_ref):
    es = jnp.concatenate([e0[...], e1[...], e2[...], e3[...]], axis=1)
    es = jnp.where(mk_ref[...] > 0.0, es, 0.0)
    y = jnp.dot(es, w_ref[...],
                preferred_element_type=jnp.float32) + bc_ref[...]
    mean = jnp.mean(y, axis=0, keepdims=True)
    var = jnp.mean((y - mean) ** 2, axis=0, keepdims=True)
    h = jnp.maximum(
        gc_ref[...] * (y - mean) / jnp.sqrt(var + EPS) + be_ref[...], 0.0)
    m = jnp.max(h, axis=1, keepdims=True)
    e = jnp.exp(h - m)
    s = e / jnp.sum(e, axis=1, keepdims=True)
    sT = s.T
    iota_f = lax.broadcasted_iota(jnp.int32, (F, B), 0)
    cntT = jnp.zeros((F, B), jnp.float32)
    for g in range(F):
      sg = sT[g:g + 1, :]
      beats = (sg > sT) | ((sg == sT) & (iota_f > g))
      cntT = cntT + jnp.where(beats, 1.0, 0.0)
    msT = jnp.where(cntT < K, sT, 0.0)
    ms_ref[...] = msT.T

  return pl.pallas_call(
      body,
      out_shape=jax.ShapeDtypeStruct((B, F), jnp.float32),
  )(*es_list, colmask, wcp, bc, gc, betac)


def _mlp(em_list, ms, expand, w1p, b1, g1, be1, w2, b2, g2, be2,
         w3, b3, g3, be3, wo, bo):
  def body(*refs):
    (em_refs, ms_ref, e_ref, w1_ref, b1_ref, g1_ref, be1_ref, w2_ref, b2_ref,
     g2_ref, be2_ref, w3_ref, b3_ref, g3_ref, be3_ref, wo_ref, bo_ref,
     o_ref, y1_s, y2_s, y3_s, s1, q1, s2, q2, s3, q3) = (
         refs[:NCB], *refs[NCB:])
    s = pl.program_id(0)
    p = s // NB
    i = s % NB

    @pl.when(p == 0)
    def _phase0():
      @pl.when(s == 0)
      def _():
        s1[...] = jnp.zeros_like(s1)
        q1[...] = jnp.zeros_like(q1)
      msx = jnp.dot(ms_ref[...], e_ref[...],
                    preferred_element_type=jnp.float32)
      z = jnp.concatenate([r[...] for r in em_refs], axis=1) * msx
      y = jnp.dot(z, w1_ref[...],
                  preferred_element_type=jnp.float32) + b1_ref[...]
      y1_s[pl.ds(i * TB, TB), :] = y
      s1[...] += jnp.sum(y, axis=0, keepdims=True)
      q1[...] += jnp.sum(y * y, axis=0, keepdims=True)

    @pl.when(p == 1)
    def _phase1():
      @pl.when(s == NB)
      def _():
        s2[...] = jnp.zeros_like(s2)
        q2[...] = jnp.zeros_like(q2)
      mean = s1[...] * (1.0 / B)
      var = q1[...] * (1.0 / B) - mean * mean
      yv = y1_s[pl.ds(i * TB, TB), :]
      h = jnp.maximum(
          g1_ref[...] * (yv - mean) / jnp.sqrt(var + EPS) + be1_ref[...], 0.0)
      y = jnp.dot(h, w2_ref[...],
                  preferred_element_type=jnp.float32) + b2_ref[...]
      y2_s[pl.ds(i * TB, TB), :] = y
      s2[...] += jnp.sum(y, axis=0, keepdims=True)
      q2[...] += jnp.sum(y * y, axis=0, keepdims=True)

    @pl.when(p == 2)
    def _phase2():
      @pl.when(s == 2 * NB)
      def _():
        s3[...] = jnp.zeros_like(s3)
        q3[...] = jnp.zeros_like(q3)
      mean = s2[...] * (1.0 / B)
      var = q2[...] * (1.0 / B) - mean * mean
      yv = y2_s[pl.ds(i * TB, TB), :]
      h = jnp.maximum(
          g2_ref[...] * (yv - mean) / jnp.sqrt(var + EPS) + be2_ref[...], 0.0)
      y = jnp.dot(h, w3_ref[...],
                  preferred_element_type=jnp.float32) + b3_ref[...]
      y3_s[pl.ds(i * TB, TB), :] = y
      s3[...] += jnp.sum(y, axis=0, keepdims=True)
      q3[...] += jnp.sum(y * y, axis=0, keepdims=True)

    @pl.when(p == 3)
    def _phase3():
      mean = s3[...] * (1.0 / B)
      var = q3[...] * (1.0 / B) - mean * mean
      yv = y3_s[pl.ds(i * TB, TB), :]
      h = jnp.maximum(
          g3_ref[...] * (yv - mean) / jnp.sqrt(var + EPS) + be3_ref[...], 0.0)
      t = jnp.dot(h, wo_ref[...],
                  preferred_element_type=jnp.float32) + bo_ref[...]
      o_ref[...] = jax.nn.sigmoid(t)

  const = lambda shape: pl.BlockSpec(shape, lambda s: (0, 0))
  tile0 = lambda shape: pl.BlockSpec(
      shape, lambda s: (jnp.minimum(s, NB - 1), 0))

  return pl.pallas_call(
      body,
      grid=(4 * NB,),
      in_specs=(
          [tile0((TB, 128)) for _ in range(NCB)] +
          [tile0((TB, F)),
           const((F, F * D)),
           const((F * D, H1)), const((1, H1)), const((1, H1)), const((1, H1)),
           const((H1, H2)), const((1, H2)), const((1, H2)), const((1, H2)),
           const((H2, H3)), const((1, H3)), const((1, H3)), const((1, H3)),
           const((H3, 1)), const((1, 1))]),
      out_specs=pl.BlockSpec(
          (TB, 1), lambda s: (jnp.where(s >= 3 * NB, s - 3 * NB, 0), 0)),
      out_shape=jax.ShapeDtypeStruct((B, 1), jnp.float32),
      scratch_shapes=[
          pltpu.VMEM((B, H1), jnp.float32),
          pltpu.VMEM((B, H2), jnp.float32),
          pltpu.VMEM((B, H3), jnp.float32),
          pltpu.VMEM((1, H1), jnp.float32), pltpu.VMEM((1, H1), jnp.float32),
          pltpu.VMEM((1, H2), jnp.float32), pltpu.VMEM((1, H2), jnp.float32),
          pltpu.VMEM((1, H3), jnp.float32), pltpu.VMEM((1, H3), jnp.float32),
      ],
  )(*em_list, ms, expand, w1p, b1, g1, be1, w2, b2, g2, be2,
    w3, b3, g3, be3, wo, bo)


def kernel(x, emb_table, emb_small_table, Wc, bc, gc, betac,
           W1, b1, g1, be1, W2, b2, g2, be2, W3, b3, g3, be3, Wo, bo):
  offs = (jnp.arange(F, dtype=jnp.int32) * PER).astype(x.dtype)
  xi_t = (x + offs[None, :]).T.astype(jnp.int32)

  outs = _sc_gather(xi_t, emb_small_table, emb_table)
  em_list, es_list = list(outs[:NCB]), list(outs[NCB:])

  c = jnp.arange(NQ * 128)
  fld = 8 * (c // 128) + (c % 128) // 16
  dsi = c % 16
  valid = fld < F
  rows = jnp.where(valid, dsi * F + jnp.minimum(fld, F - 1), 0)
  wcp = jnp.where(valid[:, None], Wc[rows, :], 0.0)
  colmask = valid.astype(jnp.float32)[None, :]
  ms = _controller(es_list, colmask, wcp, bc.reshape(1, F), gc.reshape(1, F),
                   betac.reshape(1, F))

  w1p = W1.reshape(D, F, H1).transpose(1, 0, 2).reshape(F * D, H1)
  expand = (jnp.arange(F)[:, None] ==
            (jnp.arange(F * D)[None, :] // D)).astype(jnp.float32)

  return _mlp(em_list, ms, expand, w1p, b1.reshape(1, H1),
              g1.reshape(1, H1), be1.reshape(1, H1), W2, b2.reshape(1, H2),
              g2.reshape(1, H2), be2.reshape(1, H2), W3, b3.reshape(1, H3),
              g3.reshape(1, H3), be3.reshape(1, H3), Wo, bo.reshape(1, 1))


# speedup vs baseline: 4.3179x; 1.0125x over previous
import functools

import jax
import jax.numpy as jnp
from jax import lax
from jax.experimental import pallas as pl
from jax.experimental.pallas import tpu as pltpu
from jax.experimental.pallas import tpu_sc as plsc

B = 4096
F = 26
PER = 4000
D = 64
DS = 16
K = 13
H1, H2, H3 = 1024, 512, 256
EPS = 1e-5
NCB = F // 2
NQ = 4

NC, NS = 2, 16
NW = NC * NS
BW = B // NW

TB = 512
NB = B // TB


def _sc_gather(xi_t, small_tbl, main_tbl):
  mesh = plsc.VectorSubcoreMesh(core_axis_name="c", subcore_axis_name="s")
  out_t = tuple(jax.ShapeDtypeStruct((B, 128), jnp.float32)
                for _ in range(NCB + NQ))

  @functools.partial(
      pl.kernel,
      out_type=out_t,
      mesh=mesh,
      compiler_params=pltpu.CompilerParams(use_tc_tiling_on_sc=False),
      scratch_types=[
          pltpu.VMEM((F, BW), jnp.int32),
          pltpu.VMEM((4, BW, D), jnp.float32),
          pltpu.VMEM((4, BW, DS), jnp.float32),
          pltpu.SemaphoreType.DMA((16,)),
      ],
  )
  def k(xi_hbm, sm_hbm, mn_hbm, *rest):
    outs = rest[:NCB + NQ]
    idx_all, mn_v, sm_v, sems = rest[NCB + NQ:]

    wid = lax.axis_index("s") * NC + lax.axis_index("c")
    b0 = wid * BW
    pltpu.sync_copy(xi_hbm.at[:, pl.ds(b0, BW)], idx_all)

    def fire_gathers(f):
      p = f % 4
      g1 = pltpu.async_copy(mn_hbm.at[idx_all.at[f]], mn_v.at[p],
                            sems.at[p])
      g2 = pltpu.async_copy(sm_hbm.at[idx_all.at[f]], sm_v.at[p],
                            sems.at[4 + p])
      return g1, g2

    def fire_scatters(f):
      p = f % 4
      s1 = pltpu.async_copy(
          mn_v.at[p],
          outs[f // 2].at[pl.ds(b0, BW), pl.ds(64 * (f % 2), D)],
          sems.at[8 + p])
      s2 = pltpu.async_copy(
          sm_v.at[p],
          outs[NCB + f // 8].at[pl.ds(b0, BW), pl.ds(16 * (f % 8), DS)],
          sems.at[12 + p])
      return s1, s2

    gat = {0: fire_gathers(0), 1: fire_gathers(1)}
    sca = {}
    for f in range(F):
      g1, g2 = gat.pop(f)
      g1.wait()
      g2.wait()
      sca[f] = fire_scatters(f)
      if f >= 2:
        s1, s2 = sca.pop(f - 2)
        s1.wait()
        s2.wait()
      if f + 2 < F:
        gat[f + 2] = fire_gathers(f + 2)
    for f in (F - 2, F - 1):
      s1, s2 = sca.pop(f)
      s1.wait()
      s2.wait()

  return k(xi_t, small_tbl, main_tbl)


def _controller(es_list, colmask, wcp, bc, gc, betac):
  def body(e0, e1, e2, e3, mk_ref, w_ref, bc_ref, gc_ref, be_ref, ms_ref):
    es = jnp.concatenate([e0[...], e1[...], e2[...], e3[...]], axis=1)
    es = jnp.where(mk_ref[...] > 0.0, es, 0.0)
    y = jnp.dot(es, w_ref[...],
                preferred_element_type=jnp.float32) + bc_ref[...]
    mean = jnp.mean(y, axis=0, keepdims=True)
    var = jnp.mean((y - mean) ** 2, axis=0, keepdims=True)
    h = jnp.maximum(
        gc_ref[...] * (y - mean) / jnp.sqrt(var + EPS) + be_ref[...], 0.0)
    m = jnp.max(h, axis=1, keepdims=True)
    e = jnp.exp(h - m)
    s = e / jnp.sum(e, axis=1, keepdims=True)
    sT = s.T
    iota_f = lax.broadcasted_iota(jnp.int32, (F, B), 0)
    cntT = jnp.zeros((F, B), jnp.float32)
    for g in range(F):
      sg = sT[g:g + 1, :]
      beats = (sg > sT) | ((sg == sT) & (iota_f > g))
      cntT = cntT + jnp.where(beats, 1.0, 0.0)
    msT = jnp.where(cntT < K, sT, 0.0)
    ms_ref[...] = msT.T

  return pl.pallas_call(
      body,
      out_shape=jax.ShapeDtypeStruct((B, F), jnp.float32),
  )(*es_list, colmask, wcp, bc, gc, betac)


def _mlp(em_list, ms, expand, w1p, b1, g1, be1, w2, b2, g2, be2,
         w3, b3, g3, be3, wo, bo):
  def body(*refs):
    (em_refs, ms_ref, e_ref, w1_ref, b1_ref, g1_ref, be1_ref, w2_ref, b2_ref,
     g2_ref, be2_ref, w3_ref, b3_ref, g3_ref, be3_ref, wo_ref, bo_ref,
     o_ref, y1_s, y2_s, y3_s, s1, q1, s2, q2, s3, q3) = (
         refs[:NCB], *refs[NCB:])
    s = pl.program_id(0)
    p = s // NB
    i = s % NB

    @pl.when(p == 0)
    def _phase0():
      @pl.when(s == 0)
      def _():
        s1[...] = jnp.zeros_like(s1)
        q1[...] = jnp.zeros_like(q1)
      msx = jnp.dot(ms_ref[...], e_ref[...],
                    preferred_element_type=jnp.float32)
      z = jnp.concatenate([r[...] for r in em_refs], axis=1) * msx
      y = jnp.dot(z.astype(jnp.bfloat16), w1_ref[...],
                  preferred_element_type=jnp.float32) + b1_ref[...]
      y1_s[pl.ds(i * TB, TB), :] = y
      s1[...] += jnp.sum(y, axis=0, keepdims=True)
      q1[...] += jnp.sum(y * y, axis=0, keepdims=True)

    @pl.when(p == 1)
    def _phase1():
      @pl.when(s == NB)
      def _():
        s2[...] = jnp.zeros_like(s2)
        q2[...] = jnp.zeros_like(q2)
      mean = s1[...] * (1.0 / B)
      var = q1[...] * (1.0 / B) - mean * mean
      yv = y1_s[pl.ds(i * TB, TB), :]
      h = jnp.maximum(
          g1_ref[...] * (yv - mean) / jnp.sqrt(var + EPS) + be1_ref[...], 0.0)
      y = jnp.dot(h.astype(jnp.bfloat16), w2_ref[...],
                  preferred_element_type=jnp.float32) + b2_ref[...]
      y2_s[pl.ds(i * TB, TB), :] = y
      s2[...] += jnp.sum(y, axis=0, keepdims=True)
      q2[...] += jnp.sum(y * y, axis=0, keepdims=True)

    @pl.when(p == 2)
    def _phase2():
      @pl.when(s == 2 * NB)
      def _():
        s3[...] = jnp.zeros_like(s3)
        q3[...] = jnp.zeros_like(q3)
      mean = s2[...] * (1.0 / B)
      var = q2[...] * (1.0 / B) - mean * mean
      yv = y2_s[pl.ds(i * TB, TB), :]
      h = jnp.maximum(
          g2_ref[...] * (yv - mean) / jnp.sqrt(var + EPS) + be2_ref[...], 0.0)
      y = jnp.dot(h.astype(jnp.bfloat16), w3_ref[...],
                  preferred_element_type=jnp.float32) + b3_ref[...]
      y3_s[pl.ds(i * TB, TB), :] = y
      s3[...] += jnp.sum(y, axis=0, keepdims=True)
      q3[...] += jnp.sum(y * y, axis=0, keepdims=True)

    @pl.when(p == 3)
    def _phase3():
      mean = s3[...] * (1.0 / B)
      var = q3[...] * (1.0 / B) - mean * mean
      yv = y3_s[pl.ds(i * TB, TB), :]
      h = jnp.maximum(
          g3_ref[...] * (yv - mean) / jnp.sqrt(var + EPS) + be3_ref[...], 0.0)
      t = jnp.dot(h, wo_ref[...],
                  preferred_element_type=jnp.float32) + bo_ref[...]
      o_ref[...] = jax.nn.sigmoid(t)

  const = lambda shape: pl.BlockSpec(shape, lambda s: (0, 0))
  tile0 = lambda shape: pl.BlockSpec(
      shape, lambda s: (jnp.minimum(s, NB - 1), 0))

  return pl.pallas_call(
      body,
      grid=(4 * NB,),
      in_specs=(
          [tile0((TB, 128)) for _ in range(NCB)] +
          [tile0((TB, F)),
           const((F, F * D)),
           const((F * D, H1)), const((1, H1)), const((1, H1)), const((1, H1)),
           const((H1, H2)), const((1, H2)), const((1, H2)), const((1, H2)),
           const((H2, H3)), const((1, H3)), const((1, H3)), const((1, H3)),
           const((H3, 1)), const((1, 1))]),
      out_specs=pl.BlockSpec(
          (TB, 1), lambda s: (jnp.where(s >= 3 * NB, s - 3 * NB, 0), 0)),
      out_shape=jax.ShapeDtypeStruct((B, 1), jnp.float32),
      scratch_shapes=[
          pltpu.VMEM((B, H1), jnp.float32),
          pltpu.VMEM((B, H2), jnp.float32),
          pltpu.VMEM((B, H3), jnp.float32),
          pltpu.VMEM((1, H1), jnp.float32), pltpu.VMEM((1, H1), jnp.float32),
          pltpu.VMEM((1, H2), jnp.float32), pltpu.VMEM((1, H2), jnp.float32),
          pltpu.VMEM((1, H3), jnp.float32), pltpu.VMEM((1, H3), jnp.float32),
      ],
  )(*em_list, ms, expand, w1p, b1, g1, be1, w2, b2, g2, be2,
    w3, b3, g3, be3, wo, bo)


def kernel(x, emb_table, emb_small_table, Wc, bc, gc, betac,
           W1, b1, g1, be1, W2, b2, g2, be2, W3, b3, g3, be3, Wo, bo):
  offs = (jnp.arange(F, dtype=jnp.int32) * PER).astype(x.dtype)
  xi_t = (x + offs[None, :]).T.astype(jnp.int32)

  outs = _sc_gather(xi_t, emb_small_table, emb_table)
  em_list, es_list = list(outs[:NCB]), list(outs[NCB:])

  c = jnp.arange(NQ * 128)
  fld = 8 * (c // 128) + (c % 128) // 16
  dsi = c % 16
  valid = fld < F
  rows = jnp.where(valid, dsi * F + jnp.minimum(fld, F - 1), 0)
  wcp = jnp.where(valid[:, None], Wc[rows, :], 0.0)
  colmask = valid.astype(jnp.float32)[None, :]
  ms = _controller(es_list, colmask, wcp, bc.reshape(1, F), gc.reshape(1, F),
                   betac.reshape(1, F))

  w1p = W1.reshape(D, F, H1).transpose(1, 0, 2).reshape(F * D, H1)
  expand = (jnp.arange(F)[:, None] ==
            (jnp.arange(F * D)[None, :] // D)).astype(jnp.float32)

  return _mlp(em_list, ms, expand, w1p.astype(jnp.bfloat16),
              b1.reshape(1, H1), g1.reshape(1, H1), be1.reshape(1, H1),
              W2.astype(jnp.bfloat16), b2.reshape(1, H2),
              g2.reshape(1, H2), be2.reshape(1, H2),
              W3.astype(jnp.bfloat16), b3.reshape(1, H3),
              g3.reshape(1, H3), be3.reshape(1, H3), Wo, bo.reshape(1, 1))
